# baseline (speedup 1.0000x reference)
import functools

import jax
import jax.numpy as jnp
from jax.experimental import pallas as pl
from jax.experimental.pallas import tpu as pltpu

_NBUF = 3
_TCMAX = 16384


def _chunk_schedule(rows):
    tail = [8192, 4096, 2048, 1024, 1024]
    n_big = (rows - sum(tail)) // _TCMAX
    return [_TCMAX] * n_big + tail


def _mlp_chunk(xb, w1_ref, b1_ref, w2t_ref, b2_ref, wh_ref, bh_ref):
    h = jax.lax.dot_general(
        w1_ref[...], xb, (((0,), (1,)), ((), ())),
        preferred_element_type=jnp.float32) + b1_ref[...]
    h = jnp.maximum(h, 0.0).astype(jnp.bfloat16)

    h = jnp.dot(w2t_ref[...], h,
                preferred_element_type=jnp.float32) + b2_ref[...]
    h = jnp.maximum(h, 0.0).astype(jnp.bfloat16)

    z = jnp.dot(wh_ref[...], h,
                preferred_element_type=jnp.float32) + bh_ref[...]

    zm = z[0:1, :]
    zs = z[1:2, :]
    mu = jnp.tanh(zm) * 2.0
    sig = (jnp.maximum(zs, 0.0)
           + jnp.log1p(jnp.exp(-jnp.abs(zs)))
           + 1e-5)
    return mu, sig


def _pipelined_kernel(x_hbm, w1_ref, b1_ref, w2t_ref, b2_ref, wh_ref, bh_ref,
                      mu_ref, sig_ref, x_buf, in_sem, *, rows):
    base = pl.program_id(0) * rows
    sched = _chunk_schedule(rows)
    offs = [0]
    for r in sched:
        offs.append(offs[-1] + r)

    def dma_in(slot, j):
        r = sched[j]
        pltpu.make_async_copy(
            x_hbm.at[pl.ds(base + offs[j], r)],
            x_buf.at[slot, pl.ds(0, r)], in_sem.at[slot]).start()

    def wait_in(slot, j):
        r = sched[j]
        pltpu.make_async_copy(
            x_hbm.at[pl.ds(base, r)],
            x_buf.at[slot, pl.ds(0, r)], in_sem.at[slot]).wait()

    n = len(sched)
    dma_in(0, 0)
    for j in range(n):
        cur = j % _NBUF
        if j + 1 < n:
            dma_in((j + 1) % _NBUF, j + 1)
        wait_in(cur, j)

        r, off = sched[j], offs[j]
        xb = x_buf[cur, :r].astype(jnp.bfloat16)
        mu, sig = _mlp_chunk(xb, w1_ref, b1_ref, w2t_ref, b2_ref,
                             wh_ref, bh_ref)
        mu_ref[:, off:off + r] = mu
        sig_ref[:, off:off + r] = sig


def _prep_weights(w1, b1, w2, b2, w_mu, b_mu, w_sig, b_sig):
    H = w1.shape[1]
    w1b = w1.astype(jnp.bfloat16)
    b1t = b1.reshape(H, 1)
    w2tb = w2.T.astype(jnp.bfloat16)
    b2t = b2.reshape(H, 1)
    wh = jnp.concatenate([w_mu, w_sig], axis=1).T.astype(jnp.bfloat16)
    bh = jnp.concatenate([b_mu, b_sig], axis=1).reshape(2, 1)
    return w1b, b1t, w2tb, b2t, wh, bh


def _simple_path(x, w1b, b1t, w2tb, b2t, wh, bh):
    B, S = x.shape
    H = w1b.shape[1]

    def _body(x_ref, w1_ref, b1_ref, w2t_ref, b2_ref, wh_ref, bh_ref,
              mu_ref, sig_ref):
        xb = x_ref[...].astype(jnp.bfloat16)
        mu, sig = _mlp_chunk(xb, w1_ref, b1_ref, w2t_ref, b2_ref,
                             wh_ref, bh_ref)
        mu_ref[...] = mu
        sig_ref[...] = sig

    TB = min(16384, B)
    return pl.pallas_call(
        _body,
        out_shape=(jax.ShapeDtypeStruct((1, B), jnp.float32),
                   jax.ShapeDtypeStruct((1, B), jnp.float32)),
        grid=(pl.cdiv(B, TB),),
        in_specs=[
            pl.BlockSpec((TB, S), lambda i: (i, 0)),
            pl.BlockSpec((S, H), lambda i: (0, 0)),
            pl.BlockSpec((H, 1), lambda i: (0, 0)),
            pl.BlockSpec((H, H), lambda i: (0, 0)),
            pl.BlockSpec((H, 1), lambda i: (0, 0)),
            pl.BlockSpec((2, H), lambda i: (0, 0)),
            pl.BlockSpec((2, 1), lambda i: (0, 0)),
        ],
        out_specs=(pl.BlockSpec((1, TB), lambda i: (0, i)),
                   pl.BlockSpec((1, TB), lambda i: (0, i))),
        compiler_params=pltpu.CompilerParams(
            dimension_semantics=("parallel",),
        ),
    )(x, w1b, b1t, w2tb, b2t, wh, bh)


def kernel(x, w1, b1, w2, b2, w_mu, b_mu, w_sig, b_sig):
    B, S = x.shape
    H = w1.shape[1]

    w1b, b1t, w2tb, b2t, wh, bh = _prep_weights(
        w1, b1, w2, b2, w_mu, b_mu, w_sig, b_sig)

    NP = 2

    if B % (NP * _TCMAX) != 0 or B // NP < 2 * _TCMAX:
        mu2d, sig2d = _simple_path(x, w1b, b1t, w2tb, b2t, wh, bh)
        return mu2d.reshape(B, 1), sig2d.reshape(B, 1)

    rows = B // NP

    mu2d, sig2d = pl.pallas_call(
        functools.partial(_pipelined_kernel, rows=rows),
        out_shape=(jax.ShapeDtypeStruct((1, B), jnp.float32),
                   jax.ShapeDtypeStruct((1, B), jnp.float32)),
        grid=(NP,),
        in_specs=[
            pl.BlockSpec(memory_space=pltpu.HBM),
            pl.BlockSpec((S, H), lambda i: (0, 0)),
            pl.BlockSpec((H, 1), lambda i: (0, 0)),
            pl.BlockSpec((H, H), lambda i: (0, 0)),
            pl.BlockSpec((H, 1), lambda i: (0, 0)),
            pl.BlockSpec((2, H), lambda i: (0, 0)),
            pl.BlockSpec((2, 1), lambda i: (0, 0)),
        ],
        out_specs=(pl.BlockSpec((1, rows), lambda i: (0, i)),
                   pl.BlockSpec((1, rows), lambda i: (0, i))),
        scratch_shapes=[
            pltpu.VMEM((_NBUF, _TCMAX, S), jnp.float32),
            pltpu.SemaphoreType.DMA((_NBUF,)),
        ],
        compiler_params=pltpu.CompilerParams(
            dimension_semantics=("parallel",),
        ),
    )(x, w1b, b1t, w2tb, b2t, wh, bh)

    mu = mu2d.reshape(B, 1)
    sigma = sig2d.reshape(B, 1)
    return mu, sigma


# speedup vs baseline: 1.0577x; 1.0577x over previous
import jax
import jax.numpy as jnp
from jax.experimental import pallas as pl
from jax.experimental.pallas import tpu as pltpu


def _dot0(w, a):
    return jax.lax.dot_general(w, a, (((0,), (0,)), ((), ())),
                               preferred_element_type=jnp.float32)


def _fused_policy_kernel(x_ref, w1_ref, b1_ref, w2_ref, b2_ref,
                         wmu_ref, bmu_ref, wsig_ref, bsig_ref,
                         mu_ref, sig_ref):
    xb = x_ref[...].astype(jnp.bfloat16)

    h = jax.lax.dot_general(
        w1_ref[...].astype(jnp.bfloat16), xb, (((0,), (1,)), ((), ())),
        preferred_element_type=jnp.float32) + b1_ref[...]
    h = jnp.maximum(h, 0.0).astype(jnp.bfloat16)

    h = _dot0(w2_ref[...].astype(jnp.bfloat16), h) + b2_ref[...]
    h = jnp.maximum(h, 0.0).astype(jnp.bfloat16)

    zm = _dot0(wmu_ref[...].astype(jnp.bfloat16), h) + bmu_ref[...]
    zs = _dot0(wsig_ref[...].astype(jnp.bfloat16), h) + bsig_ref[...]

    mu_ref[...] = jnp.tanh(zm) * 2.0
    sig_ref[...] = (jnp.maximum(zs, 0.0)
                    + jnp.log1p(jnp.exp(-jnp.abs(zs)))
                    + 1e-5)


def kernel(x, w1, b1, w2, b2, w_mu, b_mu, w_sig, b_sig):
    B, S = x.shape
    H = w1.shape[1]

    b1t = b1.reshape(H, 1)
    b2t = b2.reshape(H, 1)

    TB = min(16384, B)
    grid = (pl.cdiv(B, TB),)
    _const = lambda i: (0, 0)

    mu2d, sig2d = pl.pallas_call(
        _fused_policy_kernel,
        out_shape=(jax.ShapeDtypeStruct((1, B), jnp.float32),
                   jax.ShapeDtypeStruct((1, B), jnp.float32)),
        grid=grid,
        in_specs=[
            pl.BlockSpec((TB, S), lambda i: (i, 0)),
            pl.BlockSpec((S, H), _const),
            pl.BlockSpec((H, 1), _const),
            pl.BlockSpec((H, H), _const),
            pl.BlockSpec((H, 1), _const),
            pl.BlockSpec((H, 1), _const),
            pl.BlockSpec((1, 1), _const),
            pl.BlockSpec((H, 1), _const),
            pl.BlockSpec((1, 1), _const),
        ],
        out_specs=(pl.BlockSpec((1, TB), lambda i: (0, i)),
                   pl.BlockSpec((1, TB), lambda i: (0, i))),
        compiler_params=pltpu.CompilerParams(
            dimension_semantics=("parallel",),
        ),
    )(x, w1, b1t, w2, b2t, w_mu, b_mu, w_sig, b_sig)

    mu = mu2d.reshape(B, 1)
    sigma = sig2d.reshape(B, 1)
    return mu, sigma


# speedup vs baseline: 1.1493x; 1.0866x over previous
import jax
import jax.numpy as jnp
from jax.experimental import pallas as pl
from jax.experimental.pallas import tpu as pltpu


def _fused_policy_kernel(x_ref, w1_ref, b1_ref, w2t_ref, b2_ref,
                         wh_ref, bh_ref, mu_ref, sig_ref):
    xb = x_ref[...].astype(jnp.bfloat16)

    h = jax.lax.dot_general(
        w1_ref[...], xb, (((0,), (1,)), ((), ())),
        preferred_element_type=jnp.float32) + b1_ref[...]
    h = jnp.maximum(h, 0.0).astype(jnp.bfloat16)

    h = jnp.dot(w2t_ref[...], h,
                preferred_element_type=jnp.float32) + b2_ref[...]
    h = jnp.maximum(h, 0.0).astype(jnp.bfloat16)

    z = jnp.dot(wh_ref[...], h,
                preferred_element_type=jnp.float32) + bh_ref[...]

    zm = z[0:1, :]
    zs = z[1:2, :]
    mu_ref[...] = jnp.tanh(zm) * 2.0
    sig_ref[...] = (jnp.maximum(zs, 0.0)
                    + jnp.log1p(jnp.exp(-jnp.abs(zs)))
                    + 1e-5)


def kernel(x, w1, b1, w2, b2, w_mu, b_mu, w_sig, b_sig):
    B, S = x.shape
    H = w1.shape[1]

    w1b = w1.astype(jnp.bfloat16)
    b1t = b1.reshape(H, 1)
    w2tb = w2.T.astype(jnp.bfloat16)
    b2t = b2.reshape(H, 1)
    wh = jnp.concatenate([w_mu, w_sig], axis=1).T.astype(jnp.bfloat16)
    bh = jnp.concatenate([b_mu, b_sig], axis=1).reshape(2, 1)

    TB = min(16384, B)
    grid = (pl.cdiv(B, TB),)

    mu2d, sig2d = pl.pallas_call(
        _fused_policy_kernel,
        out_shape=(jax.ShapeDtypeStruct((1, B), jnp.float32),
                   jax.ShapeDtypeStruct((1, B), jnp.float32)),
        grid=grid,
        in_specs=[
            pl.BlockSpec((TB, S), lambda i: (i, 0)),
            pl.BlockSpec((S, H), lambda i: (0, 0)),
            pl.BlockSpec((H, 1), lambda i: (0, 0)),
            pl.BlockSpec((H, H), lambda i: (0, 0)),
            pl.BlockSpec((H, 1), lambda i: (0, 0)),
            pl.BlockSpec((2, H), lambda i: (0, 0)),
            pl.BlockSpec((2, 1), lambda i: (0, 0)),
        ],
        out_specs=(pl.BlockSpec((1, TB), lambda i: (0, i)),
                   pl.BlockSpec((1, TB), lambda i: (0, i))),
        compiler_params=pltpu.CompilerParams(
            dimension_semantics=("parallel",),
        ),
    )(x, w1b, b1t, w2tb, b2t, wh, bh)

    mu = mu2d.reshape(B, 1)
    sigma = sig2d.reshape(B, 1)
    return mu, sigma
